# Initial kernel scaffold; baseline (speedup 1.0000x reference)
import jax
import jax.numpy as jnp
from jax.experimental import pallas as pl


def kernel(x, edge_index, batch, W1, b1, W2, b2, W3, b3):
    raise NotImplementedError("write your pallas kernel here")


# speedup vs baseline: 8.4472x; 8.4472x over previous
import functools

import jax
import jax.numpy as jnp
from jax import lax
from jax.experimental import pallas as pl
from jax.experimental.pallas import tpu as pltpu
from jax.experimental.pallas import tpu_sc as plsc

N = 10000
NPAD = 10112
NSC = 2
NSUB = 16
NW = NSC * NSUB
CB = 128
NCHUNK = 80
GC = 8
NGRP = NCHUNK // GC
EPAD = NW * NCHUNK * CB
RPT = NPAD // NSUB
RFULL = RPT // CB
RREM = RPT - RFULL * CB
BLK = 2528


def _mesh():
    return plsc.VectorSubcoreMesh(core_axis_name="c", subcore_axis_name="s")


def _make_agg(K, npass):
    kp = K // npass

    @functools.partial(
        pl.kernel,
        out_type=jax.ShapeDtypeStruct((NSC, npass, NPAD, kp), jnp.float32),
        mesh=_mesh(),
        scratch_types=[
            pltpu.VMEM((GC, CB), jnp.int32),
            pltpu.VMEM((GC, CB), jnp.int32),
            pltpu.VMEM((CB, kp), jnp.float32),
            pltpu.VMEM((CB, kp), jnp.float32),
            pltpu.VMEM_SHARED((NPAD, kp), jnp.float32),
            pltpu.SemaphoreType.DMA,
            pltpu.SemaphoreType.DMA,
        ],
        compiler_params=pltpu.CompilerParams(use_tc_tiling_on_sc=False),
    )
    def agg(src_hbm, dst_hbm, p_hbm, out_hbm, srcv, dstv, rows0, rows1, acc,
            sem0, sem1):
        cid = lax.axis_index("c")
        sid = lax.axis_index("s")
        wid = cid * NSUB + sid
        base = sid * RPT
        for pp in range(npass):
            p_pass = p_hbm.at[pp]
            for c in range(RFULL):
                pltpu.sync_copy(p_pass.at[pl.ds(base + c * CB, CB)], rows0)
                pltpu.sync_copy(rows0, acc.at[pl.ds(base + c * CB, CB)])
            pltpu.sync_copy(p_pass.at[pl.ds(base + RFULL * CB, RREM)],
                            rows0.at[pl.ds(0, RREM)])
            pltpu.sync_copy(rows0.at[pl.ds(0, RREM)],
                            acc.at[pl.ds(base + RFULL * CB, RREM)])
            plsc.subcore_barrier()

            def group(i, carry):
                pltpu.sync_copy(src_hbm.at[wid].at[i], srcv)
                pltpu.sync_copy(dst_hbm.at[wid].at[i], dstv)
                pltpu.async_copy(p_pass.at[srcv.at[0]], rows0, sem0)
                for b in range(GC):
                    if b % 2 == 0:
                        r, s, nr, ns = rows0, sem0, rows1, sem1
                    else:
                        r, s, nr, ns = rows1, sem1, rows0, sem0
                    pltpu.make_async_copy(p_pass.at[srcv.at[b]], r, s).wait()
                    if b + 1 < GC:
                        pltpu.async_copy(p_pass.at[srcv.at[b + 1]], nr, ns)
                    pltpu.sync_copy(r, acc.at[dstv.at[b]], add=True)
                return carry

            lax.fori_loop(0, NGRP, group, 0)
            plsc.subcore_barrier()
            o_pass = out_hbm.at[cid].at[pp]
            for c in range(RFULL):
                pltpu.sync_copy(acc.at[pl.ds(base + c * CB, CB)], rows0)
                pltpu.sync_copy(rows0, o_pass.at[pl.ds(base + c * CB, CB)])
            pltpu.sync_copy(acc.at[pl.ds(base + RFULL * CB, RREM)],
                            rows0.at[pl.ds(0, RREM)])
            pltpu.sync_copy(rows0.at[pl.ds(0, RREM)],
                            o_pass.at[pl.ds(base + RFULL * CB, RREM)])
            if npass > 1 and pp + 1 < npass:
                plsc.subcore_barrier()

    return agg


_agg128 = _make_agg(128, 1)
_agg192 = _make_agg(192, 2)
_agg40 = _make_agg(40, 1)


@functools.partial(
    pl.kernel,
    out_type=jax.ShapeDtypeStruct((NSC, NPAD, 1), jnp.float32),
    mesh=_mesh(),
    scratch_types=[
        pltpu.VMEM((NCHUNK, CB), jnp.int32),
        pltpu.VMEM((CB, 1), jnp.float32),
        pltpu.VMEM((CB, 1), jnp.float32),
        pltpu.VMEM_SHARED((NPAD, 1), jnp.float32),
    ],
    compiler_params=pltpu.CompilerParams(use_tc_tiling_on_sc=False),
)
def _deg(dst_hbm, ones_hbm, zeros_hbm, out_hbm, dstv, zbuf, onesbuf, acc):
    cid = lax.axis_index("c")
    sid = lax.axis_index("s")
    wid = cid * NSUB + sid
    base = sid * RPT
    pltpu.sync_copy(dst_hbm.at[wid], dstv)
    pltpu.sync_copy(ones_hbm, onesbuf)
    pltpu.sync_copy(zeros_hbm, zbuf)
    for c in range(RFULL):
        pltpu.sync_copy(zbuf, acc.at[pl.ds(base + c * CB, CB)])
    pltpu.sync_copy(zbuf.at[pl.ds(0, RREM)],
                    acc.at[pl.ds(base + RFULL * CB, RREM)])
    plsc.subcore_barrier()

    def body(j, carry):
        pltpu.sync_copy(onesbuf, acc.at[dstv.at[j]], add=True)
        return carry

    lax.fori_loop(0, NCHUNK, body, 0)
    plsc.subcore_barrier()
    for c in range(RFULL):
        pltpu.sync_copy(acc.at[pl.ds(base + c * CB, CB)], zbuf)
        pltpu.sync_copy(zbuf, out_hbm.at[cid].at[pl.ds(base + c * CB, CB)])
    pltpu.sync_copy(acc.at[pl.ds(base + RFULL * CB, RREM)],
                    zbuf.at[pl.ds(0, RREM)])
    pltpu.sync_copy(zbuf.at[pl.ds(0, RREM)],
                    out_hbm.at[cid].at[pl.ds(base + RFULL * CB, RREM)])


def _prep1_body(deg_ref, x_ref, dinv_ref, p1_ref):
    d = deg_ref[0] + deg_ref[1] + 1.0
    dinv = lax.rsqrt(d)
    dinv_ref[...] = dinv
    p1_ref[...] = x_ref[...] * dinv


def _prep1(degp, xpad):
    return pl.pallas_call(
        _prep1_body,
        grid=(NPAD // BLK,),
        in_specs=[
            pl.BlockSpec((NSC, BLK, 1), lambda i: (0, i, 0)),
            pl.BlockSpec((BLK, 128), lambda i: (i, 0)),
        ],
        out_specs=[
            pl.BlockSpec((BLK, 1), lambda i: (i, 0)),
            pl.BlockSpec((BLK, 128), lambda i: (i, 0)),
        ],
        out_shape=[
            jax.ShapeDtypeStruct((NPAD, 1), jnp.float32),
            jax.ShapeDtypeStruct((NPAD, 128), jnp.float32),
        ],
    )(degp, xpad)


def _mid1_body(q_ref, p_ref, dinv_ref, w1_ref, b1_ref, w2_ref, t2_ref):
    agg = (q_ref[0] + q_ref[1] - p_ref[...]) * dinv_ref[...]
    h = jnp.tanh(
        jnp.dot(agg, w1_ref[...], preferred_element_type=jnp.float32)
        + b1_ref[...])
    t2_ref[...] = (
        jnp.dot(h, w2_ref[...], preferred_element_type=jnp.float32)
        * dinv_ref[...])


def _mid1(q1, p1, dinv, W1, b1, W2):
    return pl.pallas_call(
        _mid1_body,
        grid=(NPAD // BLK,),
        in_specs=[
            pl.BlockSpec((NSC, BLK, 128), lambda i: (0, i, 0)),
            pl.BlockSpec((BLK, 128), lambda i: (i, 0)),
            pl.BlockSpec((BLK, 1), lambda i: (i, 0)),
            pl.BlockSpec((128, 256), lambda i: (0, 0)),
            pl.BlockSpec((1, 256), lambda i: (0, 0)),
            pl.BlockSpec((256, 192), lambda i: (0, 0)),
        ],
        out_specs=pl.BlockSpec((BLK, 192), lambda i: (i, 0)),
        out_shape=jax.ShapeDtypeStruct((NPAD, 192), jnp.float32),
    )(q1, p1, dinv, W1, b1, W2)


def _mid2_body(q_ref, t2_ref, dinv_ref, b2_ref, w3_ref, t3_ref):
    agg = (q_ref[0] + q_ref[1] - t2_ref[...]) * dinv_ref[...]
    h = jnp.tanh(agg + b2_ref[...])
    t3_ref[...] = (
        jnp.dot(h, w3_ref[...], preferred_element_type=jnp.float32)
        * dinv_ref[...])


def _mid2(q2, t2, dinv, b2, W3):
    return pl.pallas_call(
        _mid2_body,
        grid=(NPAD // BLK,),
        in_specs=[
            pl.BlockSpec((NSC, BLK, 192), lambda i: (0, i, 0)),
            pl.BlockSpec((BLK, 192), lambda i: (i, 0)),
            pl.BlockSpec((BLK, 1), lambda i: (i, 0)),
            pl.BlockSpec((1, 192), lambda i: (0, 0)),
            pl.BlockSpec((192, 40), lambda i: (0, 0)),
        ],
        out_specs=pl.BlockSpec((BLK, 40), lambda i: (i, 0)),
        out_shape=jax.ShapeDtypeStruct((NPAD, 40), jnp.float32),
    )(q2, t2, dinv, b2, W3)


def _fin_body(q_ref, t3_ref, dinv_ref, b3_ref, out_ref):
    agg = (q_ref[0] + q_ref[1] - t3_ref[...]) * dinv_ref[...]
    out_ref[...] = jax.nn.sigmoid(agg + b3_ref[...])


def _fin(q3, t3, dinv, b3):
    return pl.pallas_call(
        _fin_body,
        grid=(NPAD // BLK,),
        in_specs=[
            pl.BlockSpec((NSC, BLK, 40), lambda i: (0, i, 0)),
            pl.BlockSpec((BLK, 40), lambda i: (i, 0)),
            pl.BlockSpec((BLK, 1), lambda i: (i, 0)),
            pl.BlockSpec((1, 40), lambda i: (0, 0)),
        ],
        out_specs=pl.BlockSpec((BLK, 40), lambda i: (i, 0)),
        out_shape=jax.ShapeDtypeStruct((NPAD, 40), jnp.float32),
    )(q3, t3, dinv, b3)


def kernel(x, edge_index, batch, W1, b1, W2, b2, W3, b3):
    del batch
    src = edge_index[0].astype(jnp.int32)
    dst = edge_index[1].astype(jnp.int32)
    npad_e = EPAD - src.shape[0]
    src_p = jnp.concatenate([src, jnp.zeros((npad_e,), jnp.int32)])
    dst_p = jnp.concatenate([dst, jnp.full((npad_e,), N, jnp.int32)])
    src_g = src_p.reshape(NW, NGRP, GC, CB)
    dst_g = dst_p.reshape(NW, NGRP, GC, CB)
    dst_f = dst_p.reshape(NW, NCHUNK, CB)
    ones_c = jnp.ones((CB, 1), jnp.float32)
    zeros_c = jnp.zeros((CB, 1), jnp.float32)

    degp = _deg(dst_f, ones_c, zeros_c)
    xpad = jnp.pad(x, ((0, NPAD - N), (0, 0)))
    dinv, p1 = _prep1(degp, xpad)
    q1 = _agg128(src_g, dst_g, p1.reshape(1, NPAD, 128))
    q1 = q1.reshape(NSC, NPAD, 128)
    t2 = _mid1(q1, p1, dinv, W1, b1.reshape(1, -1), W2)
    p2 = t2.reshape(NPAD, 2, 96).transpose(1, 0, 2)
    q2 = _agg192(src_g, dst_g, p2)
    q2 = q2.transpose(0, 2, 1, 3).reshape(NSC, NPAD, 192)
    t3 = _mid2(q2, t2, dinv, b2.reshape(1, -1), W3)
    q3 = _agg40(src_g, dst_g, t3.reshape(1, NPAD, 40))
    q3 = q3.reshape(NSC, NPAD, 40)
    outp = _fin(q3, t3, dinv, b3.reshape(1, -1))
    return outp[:N]
